# Initial kernel scaffold; baseline (speedup 1.0000x reference)
import jax
import jax.numpy as jnp
from jax.experimental import pallas as pl


def kernel(edge_uv, edge_vu, emb_user, emb_item, W1_uv_self, W1_uv_neigh, b1_uv, W1_vu_self, W1_vu_neigh, b1_vu, W2_uv_self, W2_uv_neigh, b2_uv, W2_vu_self, W2_vu_neigh, b2_vu):
    raise NotImplementedError("write your pallas kernel here")


# speedup vs baseline: 2.6088x; 2.6088x over previous
import functools

import jax
import jax.numpy as jnp
from jax import lax
from jax.experimental import pallas as pl
from jax.experimental.pallas import tpu as pltpu
from jax.experimental.pallas import tpu_sc as plsc

D = 128
NC = 2
NS = 16
NW = NC * NS
L = 16
CH = 128


def _agg_body(with_count, nchunks, rows_per_tile, n_pad,
              table, src3, dst3, *rest):
    if with_count:
        (out_p, out_c, src_v, dst_v, rows_v, ones_v, z1_v,
         acc_sh, cnt_sh, sem) = rest
    else:
        (out_p, src_v, dst_v, rows_v, ones_v, z1_v,
         acc_sh, cnt_sh, sem) = rest
        out_c = None
    c = lax.axis_index("c")
    s = lax.axis_index("s")
    wid = c * NS + s

    def _fill_rows(i, _):
        rows_v[i // (D // L), pl.ds((i % (D // L)) * L, L)] = (
            jnp.zeros((L,), jnp.float32))
        return 0
    lax.fori_loop(0, CH * (D // L), _fill_rows, 0)

    def _fill_1d(i, _):
        ones_v[pl.ds(i * L, L)] = jnp.ones((L,), jnp.float32)
        z1_v[pl.ds(i * L, L)] = jnp.zeros((L,), jnp.float32)
        return 0
    lax.fori_loop(0, CH // L, _fill_1d, 0)

    row0 = s * rows_per_tile
    for j in range(rows_per_tile // CH):
        pltpu.sync_copy(rows_v, acc_sh.at[pl.ds(row0 + j * CH, CH)])
        pltpu.sync_copy(z1_v, cnt_sh.at[pl.ds(row0 + j * CH, CH)])
    plsc.subcore_barrier()

    pltpu.sync_copy(src3.at[wid], src_v)
    pltpu.sync_copy(dst3.at[wid], dst_v)

    for k in range(nchunks):
        pltpu.async_copy(table.at[src_v.at[k]], rows_v, sem).wait()
        pltpu.sync_copy(rows_v, acc_sh.at[dst_v.at[k]], add=True)
        if with_count:
            pltpu.sync_copy(ones_v, cnt_sh.at[dst_v.at[k]], add=True)
    plsc.subcore_barrier()

    out0 = c * n_pad + row0
    pltpu.sync_copy(acc_sh.at[pl.ds(row0, rows_per_tile)],
                    out_p.at[pl.ds(out0, rows_per_tile)])
    if with_count:
        pltpu.sync_copy(cnt_sh.at[pl.ds(row0, rows_per_tile)],
                        out_c.at[pl.ds(out0, rows_per_tile)])


@functools.lru_cache(maxsize=None)
def _make_agg(n_pad, nchunks, with_count):
    rows_per_tile = n_pad // NS
    assert rows_per_tile % CH == 0
    mesh = plsc.VectorSubcoreMesh(core_axis_name="c", subcore_axis_name="s",
                                  num_cores=NC, num_subcores=NS)
    out_type = [jax.ShapeDtypeStruct((NC * n_pad, D), jnp.float32)]
    if with_count:
        out_type.append(jax.ShapeDtypeStruct((NC * n_pad,), jnp.float32))
    scratch = [
        pltpu.VMEM((nchunks, CH), jnp.int32),
        pltpu.VMEM((nchunks, CH), jnp.int32),
        pltpu.VMEM((CH, D), jnp.float32),
        pltpu.VMEM((CH,), jnp.float32),
        pltpu.VMEM((CH,), jnp.float32),
        pltpu.VMEM_SHARED((n_pad, D), jnp.float32),
        pltpu.VMEM_SHARED((n_pad,), jnp.float32),
        pltpu.SemaphoreType.DMA,
    ]
    body = functools.partial(_agg_body, with_count, nchunks, rows_per_tile,
                             n_pad)
    return pl.kernel(body, out_type=tuple(out_type), mesh=mesh,
                     scratch_types=tuple(scratch))


def _dense_body(leaky, x_ref, p0_ref, p1_ref, c0_ref, c1_ref, w_ref, b_ref,
                o_ref):
    cnt = c0_ref[:] + c1_ref[:]
    inv = 1.0 / jnp.maximum(cnt, 1.0)
    hn = (p0_ref[:] + p1_ref[:]) * inv[:, None]
    xx = jnp.concatenate([x_ref[:], hn], axis=1)
    h = jnp.dot(xx, w_ref[:], preferred_element_type=jnp.float32)
    h = h + b_ref[:]
    if leaky:
        h = jnp.where(h >= 0, h, 0.01 * h)
    o_ref[:] = h


def _dense(x, p0, p1, c0, c1, w_cat, b, leaky, block_rows=1024):
    n = x.shape[0]
    assert n % block_rows == 0
    return pl.pallas_call(
        functools.partial(_dense_body, leaky),
        grid=(n // block_rows,),
        in_specs=[
            pl.BlockSpec((block_rows, D), lambda i: (i, 0)),
            pl.BlockSpec((block_rows, D), lambda i: (i, 0)),
            pl.BlockSpec((block_rows, D), lambda i: (i, 0)),
            pl.BlockSpec((block_rows,), lambda i: (i,)),
            pl.BlockSpec((block_rows,), lambda i: (i,)),
            pl.BlockSpec((2 * D, D), lambda i: (0, 0)),
            pl.BlockSpec((D,), lambda i: (0,)),
        ],
        out_specs=pl.BlockSpec((block_rows, D), lambda i: (i, 0)),
        out_shape=jax.ShapeDtypeStruct((n, D), jnp.float32),
    )(x, p0, p1, c0, c1, w_cat, b)


def _round_up(a, m):
    return (a + m - 1) // m * m


def kernel(edge_uv, edge_vu, emb_user, emb_item,
           W1_uv_self, W1_uv_neigh, b1_uv, W1_vu_self, W1_vu_neigh, b1_vu,
           W2_uv_self, W2_uv_neigh, b2_uv, W2_vu_self, W2_vu_neigh, b2_vu):
    n_user, n_item = emb_user.shape[0], emb_item.shape[0]
    e = edge_uv.shape[1]
    n_pad = _round_up(max(n_user, n_item), NS * CH)
    e_per_tile = _round_up(-(-e // NW), CH)
    nchunks = e_per_tile // CH
    e_pad = NW * e_per_tile

    def _prep_idx(v):
        pad = jnp.full((e_pad - e,), n_pad - 1, jnp.int32)
        return jnp.concatenate([v.astype(jnp.int32), pad]).reshape(
            NW, nchunks, CH)

    su3, di3 = _prep_idx(edge_uv[0]), _prep_idx(edge_uv[1])
    si3, du3 = _prep_idx(edge_vu[0]), _prep_idx(edge_vu[1])

    xu = jnp.zeros((n_pad, D), jnp.float32).at[:n_user].set(emb_user)
    xi = jnp.zeros((n_pad, D), jnp.float32).at[:n_item].set(emb_item)

    agg_c = _make_agg(n_pad, nchunks, True)
    agg = _make_agg(n_pad, nchunks, False)

    pi, ci = agg_c(xu, su3, di3)
    pu, cu = agg_c(xi, si3, du3)
    p0i, p1i = pi[:n_pad], pi[n_pad:]
    p0u, p1u = pu[:n_pad], pu[n_pad:]
    c0i, c1i = ci[:n_pad], ci[n_pad:]
    c0u, c1u = cu[:n_pad], cu[n_pad:]

    w1_uv = jnp.concatenate([W1_uv_self, W1_uv_neigh], axis=0)
    w1_vu = jnp.concatenate([W1_vu_self, W1_vu_neigh], axis=0)
    w2_uv = jnp.concatenate([W2_uv_self, W2_uv_neigh], axis=0)
    w2_vu = jnp.concatenate([W2_vu_self, W2_vu_neigh], axis=0)

    h1_item = _dense(xi, p0i, p1i, c0i, c1i, w1_uv, b1_uv, leaky=True)
    h1_user = _dense(xu, p0u, p1u, c0u, c1u, w1_vu, b1_vu, leaky=True)

    (p2i,) = agg(h1_user, su3, di3)
    (p2u,) = agg(h1_item, si3, du3)
    h2_item = _dense(h1_item, p2i[:n_pad], p2i[n_pad:], c0i, c1i,
                     w2_uv, b2_uv, leaky=False)
    h2_user = _dense(h1_user, p2u[:n_pad], p2u[n_pad:], c0u, c1u,
                     w2_vu, b2_vu, leaky=False)
    return (h2_user[:n_user], h2_item[:n_item])


# speedup vs baseline: 3.9560x; 1.5164x over previous
import functools

import jax
import jax.numpy as jnp
from jax import lax
from jax.experimental import pallas as pl
from jax.experimental.pallas import tpu as pltpu
from jax.experimental.pallas import tpu_sc as plsc

D = 128
NC = 2
NS = 16
L = 16
CH = 128
NB = 2
IB = 8


def _agg_body(with_count, nchunks, rows_per_tile, n_pad,
              table_u, table_i, src_u3, dst_i3, src_i3, dst_u3, *rest):
    if with_count:
        out_i, out_u, out_ci, out_cu = rest[:4]
        rest = rest[4:]
    else:
        out_i, out_u = rest[:2]
        out_ci = out_cu = None
        rest = rest[2:]
    (s0_v, s1_v, d0_v, d1_v, ones_v, z1_v, acc_sh, cnt_sh) = rest[:8]
    src_v = (s0_v, s1_v)
    dst_v = (d0_v, d1_v)
    rows = rest[8:8 + NB]
    gsem = rest[8 + NB:8 + 2 * NB]
    ssem = rest[8 + 2 * NB:8 + 3 * NB]
    c = lax.axis_index("c")
    s = lax.axis_index("s")

    def _fill_rows(i, _):
        rows[0][i // (D // L), pl.ds((i % (D // L)) * L, L)] = (
            jnp.zeros((L,), jnp.float32))
        return 0
    lax.fori_loop(0, CH * (D // L), _fill_rows, 0)

    def _fill_1d(i, _):
        ones_v[pl.ds(i * L, L)] = jnp.ones((L,), jnp.float32)
        z1_v[pl.ds(i * L, L)] = jnp.zeros((L,), jnp.float32)
        return 0
    lax.fori_loop(0, CH // L, _fill_1d, 0)

    row0 = s * rows_per_tile
    for j in range(rows_per_tile // CH):
        pltpu.sync_copy(rows[0], acc_sh.at[pl.ds(row0 + j * CH, CH)])
        pltpu.sync_copy(z1_v, cnt_sh.at[pl.ds(row0 + j * CH, CH)])
    plsc.subcore_barrier()

    def run_relation(table, src3, dst3):
        def load_idx(kb):
            pltpu.sync_copy(src3.at[s, pl.ds(kb * IB, IB)], src_v[kb % 2])
            pltpu.sync_copy(dst3.at[s, pl.ds(kb * IB, IB)], dst_v[kb % 2])

        def gather(k, b):
            idx = src_v[(k // IB) % 2].at[k % IB]
            return pltpu.async_copy(table.at[idx], rows[b], gsem[b])

        def scatter(k, b):
            idx = dst_v[(k // IB) % 2].at[k % IB]
            return pltpu.async_copy(rows[b], acc_sh.at[idx], ssem[b],
                                    add=True)

        gd = [None] * NB
        sd = [None] * NB
        unwaited = set()
        load_idx(0)
        for b in range(min(NB, nchunks)):
            gd[b] = gather(b, b)
        for k in range(nchunks):
            b = k % NB
            gd[b].wait()
            if k % IB == 1 and (k // IB + 1) * IB < nchunks:
                load_idx(k // IB + 1)
            if with_count:
                idx = dst_v[(k // IB) % 2].at[k % IB]
                pltpu.sync_copy(ones_v, cnt_sh.at[idx], add=True)
            sd[b] = scatter(k, b)
            unwaited.add(b)
            pk, nk = k - 1, k - 1 + NB
            if pk >= 0 and nk < nchunks:
                pb = pk % NB
                sd[pb].wait()
                unwaited.discard(pb)
                gd[pb] = gather(nk, pb)
        for b in sorted(unwaited):
            sd[b].wait()

    @pl.when(c == 0)
    def _():
        run_relation(table_u, src_u3, dst_i3)

    @pl.when(c == 1)
    def _():
        run_relation(table_i, src_i3, dst_u3)

    plsc.subcore_barrier()

    @pl.when(c == 0)
    def _():
        pltpu.sync_copy(acc_sh.at[pl.ds(row0, rows_per_tile)],
                        out_i.at[pl.ds(row0, rows_per_tile)])
        if with_count:
            pltpu.sync_copy(cnt_sh.at[pl.ds(row0, rows_per_tile)],
                            out_ci.at[pl.ds(row0, rows_per_tile)])

    @pl.when(c == 1)
    def _():
        pltpu.sync_copy(acc_sh.at[pl.ds(row0, rows_per_tile)],
                        out_u.at[pl.ds(row0, rows_per_tile)])
        if with_count:
            pltpu.sync_copy(cnt_sh.at[pl.ds(row0, rows_per_tile)],
                            out_cu.at[pl.ds(row0, rows_per_tile)])


@functools.lru_cache(maxsize=None)
def _make_agg(n_pad, nchunks, with_count):
    rows_per_tile = n_pad // NS
    assert rows_per_tile % CH == 0
    mesh = plsc.VectorSubcoreMesh(core_axis_name="c", subcore_axis_name="s",
                                  num_cores=NC, num_subcores=NS)
    out_type = [jax.ShapeDtypeStruct((n_pad, D), jnp.float32),
                jax.ShapeDtypeStruct((n_pad, D), jnp.float32)]
    if with_count:
        out_type += [jax.ShapeDtypeStruct((n_pad,), jnp.float32),
                     jax.ShapeDtypeStruct((n_pad,), jnp.float32)]
    scratch = [
        pltpu.VMEM((IB, CH), jnp.int32),
        pltpu.VMEM((IB, CH), jnp.int32),
        pltpu.VMEM((IB, CH), jnp.int32),
        pltpu.VMEM((IB, CH), jnp.int32),
        pltpu.VMEM((CH,), jnp.float32),
        pltpu.VMEM((CH,), jnp.float32),
        pltpu.VMEM_SHARED((n_pad, D), jnp.float32),
        pltpu.VMEM_SHARED((n_pad,), jnp.float32),
    ]
    scratch += [pltpu.VMEM((CH, D), jnp.float32) for _ in range(NB)]
    scratch += [pltpu.SemaphoreType.DMA for _ in range(2 * NB)]
    body = functools.partial(_agg_body, with_count, nchunks, rows_per_tile,
                             n_pad)
    return pl.kernel(body, out_type=tuple(out_type), mesh=mesh,
                     scratch_types=tuple(scratch))


def _dense_body(leaky, x_ref, p_ref, c_ref, w_ref, b_ref, o_ref):
    inv = 1.0 / jnp.maximum(c_ref[:], 1.0)
    hn = p_ref[:] * inv[:, None]
    xx = jnp.concatenate([x_ref[:], hn], axis=1)
    h = jnp.dot(xx, w_ref[:], preferred_element_type=jnp.float32)
    h = h + b_ref[:]
    if leaky:
        h = jnp.where(h >= 0, h, 0.01 * h)
    o_ref[:] = h


def _dense(x, p, cnt, w_cat, b, leaky, block_rows=1024):
    n = x.shape[0]
    assert n % block_rows == 0
    return pl.pallas_call(
        functools.partial(_dense_body, leaky),
        grid=(n // block_rows,),
        in_specs=[
            pl.BlockSpec((block_rows, D), lambda i: (i, 0)),
            pl.BlockSpec((block_rows, D), lambda i: (i, 0)),
            pl.BlockSpec((block_rows,), lambda i: (i,)),
            pl.BlockSpec((2 * D, D), lambda i: (0, 0)),
            pl.BlockSpec((D,), lambda i: (0,)),
        ],
        out_specs=pl.BlockSpec((block_rows, D), lambda i: (i, 0)),
        out_shape=jax.ShapeDtypeStruct((n, D), jnp.float32),
    )(x, p, cnt, w_cat, b)


def _round_up(a, m):
    return (a + m - 1) // m * m


def kernel(edge_uv, edge_vu, emb_user, emb_item,
           W1_uv_self, W1_uv_neigh, b1_uv, W1_vu_self, W1_vu_neigh, b1_vu,
           W2_uv_self, W2_uv_neigh, b2_uv, W2_vu_self, W2_vu_neigh, b2_vu):
    n_user, n_item = emb_user.shape[0], emb_item.shape[0]
    e = edge_uv.shape[1]
    n_pad = _round_up(max(n_user, n_item), NS * CH)
    e_per_tile = _round_up(-(-e // NS), CH * IB)
    nchunks = e_per_tile // CH
    e_pad = NS * e_per_tile

    def _prep_idx(v):
        pad = jnp.full((e_pad - e,), n_pad - 1, jnp.int32)
        return jnp.concatenate([v.astype(jnp.int32), pad]).reshape(
            NS, nchunks, CH)

    su3, di3 = _prep_idx(edge_uv[0]), _prep_idx(edge_uv[1])
    si3, du3 = _prep_idx(edge_vu[0]), _prep_idx(edge_vu[1])

    xu = jnp.zeros((n_pad, D), jnp.float32).at[:n_user].set(emb_user)
    xi = jnp.zeros((n_pad, D), jnp.float32).at[:n_item].set(emb_item)

    agg_c = _make_agg(n_pad, nchunks, True)
    agg = _make_agg(n_pad, nchunks, False)

    p1i, p1u, ci, cu = agg_c(xu, xi, su3, di3, si3, du3)

    w1_uv = jnp.concatenate([W1_uv_self, W1_uv_neigh], axis=0)
    w1_vu = jnp.concatenate([W1_vu_self, W1_vu_neigh], axis=0)
    w2_uv = jnp.concatenate([W2_uv_self, W2_uv_neigh], axis=0)
    w2_vu = jnp.concatenate([W2_vu_self, W2_vu_neigh], axis=0)

    h1_item = _dense(xi, p1i, ci, w1_uv, b1_uv, leaky=True)
    h1_user = _dense(xu, p1u, cu, w1_vu, b1_vu, leaky=True)

    p2i, p2u = agg(h1_user, h1_item, su3, di3, si3, du3)
    h2_item = _dense(h1_item, p2i, ci, w2_uv, b2_uv, leaky=False)
    h2_user = _dense(h1_user, p2u, cu, w2_vu, b2_vu, leaky=False)
    return (h2_user[:n_user], h2_item[:n_item])


# speedup vs baseline: 4.5758x; 1.1567x over previous
import functools

import jax
import jax.numpy as jnp
from jax import lax
from jax.experimental import pallas as pl
from jax.experimental.pallas import tpu as pltpu
from jax.experimental.pallas import tpu_sc as plsc

D = 128
NC = 2
NS = 16
L = 16
CH = 64
NB = 5
IB = 8


def _agg_body(with_count, nchunks, rows_per_tile, n_pad,
              table_u, table_i, src_u3, dst_i3, src_i3, dst_u3,
              zrows_h, zcnt_h, ones_h, *rest):
    if with_count:
        out_i, out_u, out_ci, out_cu = rest[:4]
        rest = rest[4:]
    else:
        out_i, out_u = rest[:2]
        out_ci = out_cu = None
        rest = rest[2:]
    (s0_v, s1_v, d0_v, d1_v, ones_v, z1_v, acc_sh, cnt_sh) = rest[:8]
    src_v = (s0_v, s1_v)
    dst_v = (d0_v, d1_v)
    rows = rest[8:8 + NB]
    gsem = rest[8 + NB:8 + 2 * NB]
    ssem = rest[8 + 2 * NB:8 + 3 * NB]
    c = lax.axis_index("c")
    s = lax.axis_index("s")

    pltpu.sync_copy(zrows_h, rows[0])
    pltpu.sync_copy(ones_h, ones_v)
    pltpu.sync_copy(zcnt_h, z1_v)

    row0 = pl.multiple_of(s * rows_per_tile, rows_per_tile)
    for j in range(rows_per_tile // CH):
        pltpu.sync_copy(rows[0], acc_sh.at[pl.ds(row0 + j * CH, CH)])
        pltpu.sync_copy(z1_v, cnt_sh.at[pl.ds(row0 + j * CH, CH)])
    plsc.subcore_barrier()

    def run_relation(table, src3, dst3):
        def load_idx(kb):
            pltpu.sync_copy(src3.at[s, pl.ds(kb * IB, IB)], src_v[kb % 2])
            pltpu.sync_copy(dst3.at[s, pl.ds(kb * IB, IB)], dst_v[kb % 2])

        def gather(k, b):
            idx = src_v[(k // IB) % 2].at[k % IB]
            return pltpu.async_copy(table.at[idx], rows[b], gsem[b])

        def scatter(k, b):
            idx = dst_v[(k // IB) % 2].at[k % IB]
            return pltpu.async_copy(rows[b], acc_sh.at[idx], ssem[b],
                                    add=True)

        gd = [None] * NB
        sd = [None] * NB
        unwaited = set()
        load_idx(0)
        for b in range(min(NB, nchunks)):
            gd[b] = gather(b, b)
        for k in range(nchunks):
            b = k % NB
            gd[b].wait()
            if k % IB == 1 and (k // IB + 1) * IB < nchunks:
                load_idx(k // IB + 1)
            if with_count:
                idx = dst_v[(k // IB) % 2].at[k % IB]
                pltpu.sync_copy(ones_v, cnt_sh.at[idx], add=True)
            sd[b] = scatter(k, b)
            unwaited.add(b)
            pk, nk = k - 1, k - 1 + NB
            if pk >= 0 and nk < nchunks:
                pb = pk % NB
                sd[pb].wait()
                unwaited.discard(pb)
                gd[pb] = gather(nk, pb)
        for b in sorted(unwaited):
            sd[b].wait()

    @pl.when(c == 0)
    def _():
        run_relation(table_u, src_u3, dst_i3)

    @pl.when(c == 1)
    def _():
        run_relation(table_i, src_i3, dst_u3)

    plsc.subcore_barrier()

    @pl.when(c == 0)
    def _():
        pltpu.sync_copy(acc_sh.at[pl.ds(row0, rows_per_tile)],
                        out_i.at[pl.ds(row0, rows_per_tile)])
        if with_count:
            pltpu.sync_copy(cnt_sh.at[pl.ds(row0, rows_per_tile)],
                            out_ci.at[pl.ds(row0, rows_per_tile)])

    @pl.when(c == 1)
    def _():
        pltpu.sync_copy(acc_sh.at[pl.ds(row0, rows_per_tile)],
                        out_u.at[pl.ds(row0, rows_per_tile)])
        if with_count:
            pltpu.sync_copy(cnt_sh.at[pl.ds(row0, rows_per_tile)],
                            out_cu.at[pl.ds(row0, rows_per_tile)])


@functools.lru_cache(maxsize=None)
def _make_agg(n_pad, nchunks, with_count):
    rows_per_tile = n_pad // NS
    assert rows_per_tile % CH == 0
    mesh = plsc.VectorSubcoreMesh(core_axis_name="c", subcore_axis_name="s",
                                  num_cores=NC, num_subcores=NS)
    out_type = [jax.ShapeDtypeStruct((n_pad, D), jnp.float32),
                jax.ShapeDtypeStruct((n_pad, D), jnp.float32)]
    if with_count:
        out_type += [jax.ShapeDtypeStruct((n_pad,), jnp.float32),
                     jax.ShapeDtypeStruct((n_pad,), jnp.float32)]
    scratch = [
        pltpu.VMEM((IB, CH), jnp.int32),
        pltpu.VMEM((IB, CH), jnp.int32),
        pltpu.VMEM((IB, CH), jnp.int32),
        pltpu.VMEM((IB, CH), jnp.int32),
        pltpu.VMEM((CH,), jnp.float32),
        pltpu.VMEM((CH,), jnp.float32),
        pltpu.VMEM_SHARED((n_pad, D), jnp.float32),
        pltpu.VMEM_SHARED((n_pad,), jnp.float32),
    ]
    scratch += [pltpu.VMEM((CH, D), jnp.float32) for _ in range(NB)]
    scratch += [pltpu.SemaphoreType.DMA for _ in range(2 * NB)]
    body = functools.partial(_agg_body, with_count, nchunks, rows_per_tile,
                             n_pad)
    return pl.kernel(body, out_type=tuple(out_type), mesh=mesh,
                     scratch_types=tuple(scratch))


def _dense_body(leaky, emit_bf16, x_ref, p_ref, c_ref, w_ref, b_ref, *o_refs):
    inv = 1.0 / jnp.maximum(c_ref[:], 1.0)
    hn = p_ref[:].astype(jnp.float32) * inv[:, None]
    xx = jnp.concatenate([x_ref[:], hn], axis=1)
    h = jnp.dot(xx, w_ref[:], preferred_element_type=jnp.float32)
    h = h + b_ref[:]
    if leaky:
        h = jnp.where(h >= 0, h, 0.01 * h)
    o_refs[0][...] = h
    if emit_bf16:
        o_refs[1][...] = h.astype(jnp.bfloat16)


def _dense(x, p, cnt, w_cat, b, leaky, emit_bf16, block_rows=1024):
    n = x.shape[0]
    assert n % block_rows == 0
    out_shape = [jax.ShapeDtypeStruct((n, D), jnp.float32)]
    out_specs = [pl.BlockSpec((block_rows, D), lambda i: (i, 0))]
    if emit_bf16:
        out_shape.append(jax.ShapeDtypeStruct((n, D), jnp.bfloat16))
        out_specs.append(pl.BlockSpec((block_rows, D), lambda i: (i, 0)))
    return pl.pallas_call(
        functools.partial(_dense_body, leaky, emit_bf16),
        grid=(n // block_rows,),
        in_specs=[
            pl.BlockSpec((block_rows, D), lambda i: (i, 0)),
            pl.BlockSpec((block_rows, D), lambda i: (i, 0)),
            pl.BlockSpec((block_rows,), lambda i: (i,)),
            pl.BlockSpec((2 * D, D), lambda i: (0, 0)),
            pl.BlockSpec((D,), lambda i: (0,)),
        ],
        out_specs=out_specs,
        out_shape=out_shape,
    )(x, p, cnt, w_cat, b)


def _round_up(a, m):
    return (a + m - 1) // m * m


def kernel(edge_uv, edge_vu, emb_user, emb_item,
           W1_uv_self, W1_uv_neigh, b1_uv, W1_vu_self, W1_vu_neigh, b1_vu,
           W2_uv_self, W2_uv_neigh, b2_uv, W2_vu_self, W2_vu_neigh, b2_vu):
    n_user, n_item = emb_user.shape[0], emb_item.shape[0]
    e = edge_uv.shape[1]
    n_pad = _round_up(max(n_user, n_item), NS * CH)
    e_per_tile = _round_up(-(-e // NS), CH * IB)
    nchunks = e_per_tile // CH
    e_pad = NS * e_per_tile

    def _prep_idx(v):
        pad = jnp.full((e_pad - e,), n_pad - 1, jnp.int32)
        return jnp.concatenate([v.astype(jnp.int32), pad]).reshape(
            NS, nchunks, CH)

    su3, di3 = _prep_idx(edge_uv[0]), _prep_idx(edge_uv[1])
    si3, du3 = _prep_idx(edge_vu[0]), _prep_idx(edge_vu[1])

    xu = jnp.zeros((n_pad, D), jnp.float32).at[:n_user].set(emb_user)
    xi = jnp.zeros((n_pad, D), jnp.float32).at[:n_item].set(emb_item)
    agg_c = _make_agg(n_pad, nchunks, True)
    agg = _make_agg(n_pad, nchunks, False)

    zrows = jnp.zeros((CH, D), jnp.float32)
    zcnt = jnp.zeros((CH,), jnp.float32)
    ones = jnp.ones((CH,), jnp.float32)

    p1i, p1u, ci, cu = agg_c(xu, xi, su3, di3, si3, du3,
                             zrows, zcnt, ones)

    w1_uv = jnp.concatenate([W1_uv_self, W1_uv_neigh], axis=0)
    w1_vu = jnp.concatenate([W1_vu_self, W1_vu_neigh], axis=0)
    w2_uv = jnp.concatenate([W2_uv_self, W2_uv_neigh], axis=0)
    w2_vu = jnp.concatenate([W2_vu_self, W2_vu_neigh], axis=0)

    (h1_item,) = _dense(xi, p1i, ci, w1_uv, b1_uv,
                        leaky=True, emit_bf16=False)
    (h1_user,) = _dense(xu, p1u, cu, w1_vu, b1_vu,
                        leaky=True, emit_bf16=False)

    p2i, p2u = agg(h1_user, h1_item, su3, di3, si3, du3,
                   zrows, zcnt, ones)
    (h2_item,) = _dense(h1_item, p2i, ci, w2_uv, b2_uv,
                        leaky=False, emit_bf16=False)
    (h2_user,) = _dense(h1_user, p2u, cu, w2_vu, b2_vu,
                        leaky=False, emit_bf16=False)
    return (h2_user[:n_user], h2_item[:n_item])
